# baseline (speedup 1.0000x reference)
import functools

import jax
import jax.numpy as jnp
import numpy as np
from jax import lax
from jax.experimental import pallas as pl
from jax.experimental.pallas import tpu as pltpu
from jax.experimental.pallas import tpu_sc as plsc

NLEV = 16
TSIZE_MASK = 262143
DENSE0 = 35937
LEVSZ = 262144
TOTAL = DENSE0 + 15 * LEVSZ
BODY = (TOTAL // 128) * 128
TILES = BODY // 128
TOTALP = BODY + 128
SUP = 32
SUPN = (TILES + SUP - 1) // SUP
SPS = (SUPN + 15) // 16
P1 = int(np.uint32(2654435761).astype(np.int32))
P2 = 805459861
PROJ = ((0, 1, 2), (0, 1, 3), (1, 2, 3), (0, 2, 3))

NC, NS = 2, 16
NW = NC * NS
LANES = 16
C = 128
G = C // LANES
NR = NLEV * 8 * C
OCOLS = 4 * NLEV * 2


def _corner_rows(l, tb, px0, px1, py0, py1, pz0, pz1, dense):
    hx = (px0, px1)
    if dense:
        iy = (py0 * 33, py1 * 33)
        iz = (pz0 * 1089, pz1 * 1089)
        return [hx[kx] + iy[ky] + iz[kz] + tb
                for kx in (0, 1) for ky in (0, 1) for kz in (0, 1)]
    off = tb + ((DENSE0 - LEVSZ) + l * LEVSZ)
    hy0 = py0 * jnp.int32(P1)
    hy = (hy0, hy0 + jnp.int32(P1))
    hz0 = pz0 * jnp.int32(P2)
    hz = (hz0, hz0 + jnp.int32(P2))
    a = [hx[kx] ^ hy[ky] for kx in (0, 1) for ky in (0, 1)]
    return [((a[kx * 2 + ky] ^ hz[kz]) & jnp.int32(TSIZE_MASK)) + off
            for kx in (0, 1) for ky in (0, 1) for kz in (0, 1)]


@functools.lru_cache(maxsize=None)
def _build(n):
    assert n % (NW * C) == 0 and (NW * C) % 128 == 0
    chunks = n // (NW * C)
    mesh = plsc.VectorSubcoreMesh(core_axis_name="c", subcore_axis_name="s")

    @functools.partial(
        pl.kernel,
        out_type=(jax.ShapeDtypeStruct((n * OCOLS,), jnp.float32),
                  jax.ShapeDtypeStruct((2 * 4 * TOTALP,), jnp.int32)),
        mesh=mesh,
        scratch_types=[
            pltpu.VMEM((2 * 512,), jnp.float32),
            pltpu.VMEM((NR,), jnp.int32),
            pltpu.VMEM((NR,), jnp.int32),
            pltpu.VMEM((NR,), jnp.float32),
            pltpu.VMEM((NR,), jnp.float32),
            pltpu.VMEM((NR,), jnp.int32),
            pltpu.VMEM((NR,), jnp.int32),
            pltpu.VMEM((C * OCOLS,), jnp.float32),
            pltpu.VMEM((SUP * 256,), jnp.float32),
            pltpu.VMEM((SUP * 128,), jnp.int32),
            pltpu.SemaphoreType.DMA,
            pltpu.SemaphoreType.DMA,
        ],
        compiler_params=pltpu.CompilerParams(needs_layout_passes=False),
    )
    def sc_encode(coords_hbm, b0, b1, b2, b3, e0, e1, e2, e3,
                  out_hbm, scr,
                  cbuf, ix0, ix1, w0, w1, rw0, rw1, obuf, fvbuf, pkbuf,
                  sem0, sem1):
        cid = lax.axis_index("c")
        sid = lax.axis_index("s")
        wid = sid * NC + cid
        bodies = (b0, b1, b2, b3)
        tails = (e0, e1, e2, e3)
        ixs, ws, rws, sems = (ix0, ix1), (w0, w1), (rw0, rw1), (sem0, sem1)
        iota = lax.iota(jnp.int32, LANES)
        srow = iota * OCOLS

        def pack16(a, b):
            return plsc.bitcast(
                plsc.pack(a, b, format=plsc.PackFormat.INTERLEAVED), jnp.int32)

        for t in range(4):
            tb = (cid * 4 + t) * TOTALP

            @pl.loop(0, SPS)
            def _sup(j):
                q = jnp.minimum(sid * SPS + j, SUPN - 1)
                t0 = jnp.minimum(q * SUP, TILES - SUP)
                pltpu.sync_copy(bodies[t].at[pl.ds(t0 * 256, SUP * 256)], fvbuf)

                @pl.loop(0, SUP)
                def _tile(j2):
                    for j3 in range(8):
                        a = fvbuf[pl.ds(j2 * 256 + j3 * LANES, LANES)]
                        b = fvbuf[pl.ds(j2 * 256 + 128 + j3 * LANES, LANES)]
                        pkbuf[pl.ds(j2 * 128 + j3 * LANES, LANES)] = pack16(a, b)

                pltpu.sync_copy(pkbuf, scr.at[pl.ds(tb + t0 * 128, SUP * 128)])

            @pl.when(sid == 0)
            def _tail():
                pltpu.sync_copy(tails[t], fvbuf.at[pl.ds(0, 256)])
                for j3 in range(8):
                    a = fvbuf[pl.ds(j3 * LANES, LANES)]
                    b = fvbuf[pl.ds(128 + j3 * LANES, LANES)]
                    pkbuf[pl.ds(j3 * LANES, LANES)] = pack16(a, b)
                pltpu.sync_copy(pkbuf.at[pl.ds(0, 128)],
                                scr.at[pl.ds(tb + BODY, 128)])

        plsc.subcore_barrier()

        def load_coords(ch, par):
            p0 = jnp.minimum((wid * chunks + ch) * C, n - C)
            pltpu.sync_copy(coords_hbm.at[pl.ds((p0 >> 7) * 512, 512)],
                            cbuf.at[pl.ds(par * 512, 512)])

        def coff(ch):
            p0 = (wid * chunks + ch) * C
            return (ch & 1) * 512 + (p0 & 127)

        def phase1(t, l, cb, ixb, wb_ref, dense):
            d0, d1, d2 = PROJ[t]
            tb = (cid * 4 + t) * TOTALP
            r = jnp.int32(32) << l
            rf = r.astype(jnp.float32)
            for g in range(G):
                co = cb + g * LANES
                x = cbuf[pl.ds(d0 * 128 + co, LANES)]
                y = cbuf[pl.ds(d1 * 128 + co, LANES)]
                z = cbuf[pl.ds(d2 * 128 + co, LANES)]
                sx, sy, sz = x * rf, y * rf, z * rf
                px0 = sx.astype(jnp.int32)
                py0 = sy.astype(jnp.int32)
                pz0 = sz.astype(jnp.int32)
                fx = sx - px0.astype(jnp.float32)
                fy = sy - py0.astype(jnp.float32)
                fz = sz - pz0.astype(jnp.float32)
                rows = _corner_rows(l, tb, px0, px0 + 1, py0, py0 + 1,
                                    pz0, pz0 + 1, dense)
                wx = (1.0 - fx, fx)
                wy = (1.0 - fy, fy)
                wz = (1.0 - fz, fz)
                wxy = [wx[kx] * wy[ky] for kx in (0, 1) for ky in (0, 1)]
                wb = l * (8 * C) + g * LANES
                for k in range(8):
                    kx, ky, kz = (k >> 2) & 1, (k >> 1) & 1, k & 1
                    ixb[pl.ds(wb + k * C, LANES)] = rows[k]
                    wb_ref[pl.ds(wb + k * C, LANES)] = wxy[kx * 2 + ky] * wz[kz]

        def build_and_fire(t, cb, p):
            phase1(t, 0, cb, ixs[p], ws[p], True)

            @pl.loop(1, NLEV)
            def _lev(l):
                phase1(t, l, cb, ixs[p], ws[p], False)
            pltpu.async_copy(scr.at[ixs[p]], rws[p], sems[p])

        def combine(t, p):
            pltpu.make_async_copy(scr.at[ixs[p]], rws[p], sems[p]).wait()

            @pl.loop(0, NLEV)
            def _lev2(l):
                for g in range(G):
                    acc0 = jnp.zeros((LANES,), jnp.float32)
                    acc1 = jnp.zeros((LANES,), jnp.float32)
                    wb = l * (8 * C) + g * LANES
                    for k in range(8):
                        u = rws[p][pl.ds(wb + k * C, LANES)]
                        v0 = plsc.bitcast(u << 16, jnp.float32)
                        v1 = plsc.bitcast(u & jnp.int32(-65536), jnp.float32)
                        wv = ws[p][pl.ds(wb + k * C, LANES)]
                        acc0 = acc0 + v0 * wv
                        acc1 = acc1 + v1 * wv
                    svec = srow + (g * LANES * OCOLS + t * 2 * NLEV) + l * 2
                    plsc.store_scatter(obuf, [svec], acc0)
                    plsc.store_scatter(obuf, [svec + 1], acc1)

        load_coords(0, 0)
        build_and_fire(0, coff(0), 0)

        @pl.loop(0, chunks)
        def _chunk(ch):
            load_coords(ch + 1, (ch + 1) & 1)
            for t in range(4):
                p = t & 1
                np_ = (t + 1) & 1
                if t < 3:
                    build_and_fire(t + 1, coff(ch), np_)
                else:
                    build_and_fire(0, coff(ch + 1), np_)
                combine(t, p)
            base = wid * chunks + ch
            pltpu.sync_copy(obuf, out_hbm.at[pl.ds(base * (C * OCOLS), C * OCOLS)])

        pltpu.make_async_copy(scr.at[ix0], rw0, sem0).wait()

    return sc_encode


def _body_view(t):
    return t[:BODY].reshape(BODY // 128, 128, 2).transpose(0, 2, 1).reshape(-1)


def _tail_tile(t):
    return jnp.pad(t[BODY:], ((0, 128 - (TOTAL - BODY)), (0, 0))).T.reshape(-1)


def kernel(coords, table_xyz, table_xyt, table_yzt, table_xzt):
    n = coords.shape[0]
    ts = (table_xyz, table_xyt, table_yzt, table_xzt)
    coords_v = coords.reshape(n // 128, 128, 4).transpose(0, 2, 1).reshape(-1)
    out, _ = _build(n)(coords_v, *[_body_view(t) for t in ts],
                       *[_tail_tile(t) for t in ts])
    return out.reshape(n, OCOLS)


# speedup vs baseline: 1.0343x; 1.0343x over previous
import functools

import jax
import jax.numpy as jnp
import numpy as np
from jax import lax
from jax.experimental import pallas as pl
from jax.experimental.pallas import tpu as pltpu
from jax.experimental.pallas import tpu_sc as plsc

NLEV = 16
TSIZE_MASK = 262143
DENSE0 = 35937
LEVSZ = 262144
TOTAL = DENSE0 + 15 * LEVSZ
BODY = (TOTAL // 128) * 128
TILES = BODY // 128
TOTALP = BODY + 128
SUP = 64
SUPN = (TILES + SUP - 1) // SUP
SPS = (SUPN + 15) // 16
P1 = int(np.uint32(2654435761).astype(np.int32))
P2 = 805459861
PROJ = ((0, 1, 2), (0, 1, 3), (1, 2, 3), (0, 2, 3))

NC, NS = 2, 16
NW = NC * NS
LANES = 16
C = 64
G = C // LANES
NR = NLEV * 8 * C
OCOLS = 4 * NLEV * 2


def _corner_rows(l, tb, px0, px1, py0, py1, pz0, pz1, dense):
    hx = (px0, px1)
    if dense:
        iy = (py0 * 33, py1 * 33)
        iz = (pz0 * 1089, pz1 * 1089)
        return [hx[kx] + iy[ky] + iz[kz] + tb
                for kx in (0, 1) for ky in (0, 1) for kz in (0, 1)]
    off = tb + ((DENSE0 - LEVSZ) + l * LEVSZ)
    hy0 = py0 * jnp.int32(P1)
    hy = (hy0, hy0 + jnp.int32(P1))
    hz0 = pz0 * jnp.int32(P2)
    hz = (hz0, hz0 + jnp.int32(P2))
    a = [hx[kx] ^ hy[ky] for kx in (0, 1) for ky in (0, 1)]
    return [((a[kx * 2 + ky] ^ hz[kz]) & jnp.int32(TSIZE_MASK)) + off
            for kx in (0, 1) for ky in (0, 1) for kz in (0, 1)]


@functools.lru_cache(maxsize=None)
def _build(n):
    assert n % (NW * C) == 0 and (NW * C) % 128 == 0
    chunks = n // (NW * C)
    mesh = plsc.VectorSubcoreMesh(core_axis_name="c", subcore_axis_name="s")

    @functools.partial(
        pl.kernel,
        out_type=(jax.ShapeDtypeStruct((n * OCOLS,), jnp.float32),
                  jax.ShapeDtypeStruct((2 * 4 * TOTALP,), jnp.int32)),
        mesh=mesh,
        scratch_types=[
            pltpu.VMEM((2 * 512,), jnp.float32),
            pltpu.VMEM((NR,), jnp.int32),
            pltpu.VMEM((NR,), jnp.int32),
            pltpu.VMEM((NR,), jnp.float32),
            pltpu.VMEM((NR,), jnp.float32),
            pltpu.VMEM((NR,), jnp.int32),
            pltpu.VMEM((NR,), jnp.int32),
            pltpu.VMEM((C * OCOLS,), jnp.float32),
            pltpu.VMEM((SUP * 256,), jnp.float32),
            pltpu.VMEM((SUP * 128,), jnp.int32),
            pltpu.SemaphoreType.DMA,
            pltpu.SemaphoreType.DMA,
        ],
        compiler_params=pltpu.CompilerParams(needs_layout_passes=False),
    )
    def sc_encode(coords_hbm, b0, b1, b2, b3, e0, e1, e2, e3,
                  out_hbm, scr,
                  cbuf, ix0, ix1, w0, w1, rw0, rw1, obuf, fvbuf, pkbuf,
                  sem0, sem1):
        cid = lax.axis_index("c")
        sid = lax.axis_index("s")
        wid = sid * NC + cid
        bodies = (b0, b1, b2, b3)
        tails = (e0, e1, e2, e3)
        ixs, ws, rws, sems = (ix0, ix1), (w0, w1), (rw0, rw1), (sem0, sem1)
        iota = lax.iota(jnp.int32, LANES)
        srow = iota * OCOLS

        def pack16(a, b):
            return plsc.bitcast(
                plsc.pack(a, b, format=plsc.PackFormat.INTERLEAVED), jnp.int32)

        for t in range(4):
            tb = (cid * 4 + t) * TOTALP

            @pl.loop(0, SPS)
            def _sup(j):
                q = jnp.minimum(sid * SPS + j, SUPN - 1)
                t0 = jnp.minimum(q * SUP, TILES - SUP)
                pltpu.sync_copy(bodies[t].at[pl.ds(t0 * 256, SUP * 256)], fvbuf)

                @pl.loop(0, SUP)
                def _tile(j2):
                    for j3 in range(8):
                        a = fvbuf[pl.ds(j2 * 256 + j3 * LANES, LANES)]
                        b = fvbuf[pl.ds(j2 * 256 + 128 + j3 * LANES, LANES)]
                        pkbuf[pl.ds(j2 * 128 + j3 * LANES, LANES)] = pack16(a, b)

                pltpu.sync_copy(pkbuf, scr.at[pl.ds(tb + t0 * 128, SUP * 128)])

            @pl.when(sid == 0)
            def _tail():
                pltpu.sync_copy(tails[t], fvbuf.at[pl.ds(0, 256)])
                for j3 in range(8):
                    a = fvbuf[pl.ds(j3 * LANES, LANES)]
                    b = fvbuf[pl.ds(128 + j3 * LANES, LANES)]
                    pkbuf[pl.ds(j3 * LANES, LANES)] = pack16(a, b)
                pltpu.sync_copy(pkbuf.at[pl.ds(0, 128)],
                                scr.at[pl.ds(tb + BODY, 128)])

        plsc.subcore_barrier()

        def load_coords(ch, par):
            p0 = jnp.minimum((wid * chunks + ch) * C, n - C)
            pltpu.sync_copy(coords_hbm.at[pl.ds((p0 >> 7) * 512, 512)],
                            cbuf.at[pl.ds(par * 512, 512)])

        def coff(ch):
            p0 = (wid * chunks + ch) * C
            return (ch & 1) * 512 + (p0 & 127)

        def phase1(t, l, cb, ixb, wb_ref, dense):
            d0, d1, d2 = PROJ[t]
            tb = (cid * 4 + t) * TOTALP
            r = jnp.int32(32) << l
            rf = r.astype(jnp.float32)
            for g in range(G):
                co = cb + g * LANES
                x = cbuf[pl.ds(d0 * 128 + co, LANES)]
                y = cbuf[pl.ds(d1 * 128 + co, LANES)]
                z = cbuf[pl.ds(d2 * 128 + co, LANES)]
                sx, sy, sz = x * rf, y * rf, z * rf
                px0 = sx.astype(jnp.int32)
                py0 = sy.astype(jnp.int32)
                pz0 = sz.astype(jnp.int32)
                fx = sx - px0.astype(jnp.float32)
                fy = sy - py0.astype(jnp.float32)
                fz = sz - pz0.astype(jnp.float32)
                rows = _corner_rows(l, tb, px0, px0 + 1, py0, py0 + 1,
                                    pz0, pz0 + 1, dense)
                wx = (1.0 - fx, fx)
                wy = (1.0 - fy, fy)
                wz = (1.0 - fz, fz)
                wxy = [wx[kx] * wy[ky] for kx in (0, 1) for ky in (0, 1)]
                wb = l * (8 * C) + g * LANES
                for k in range(8):
                    kx, ky, kz = (k >> 2) & 1, (k >> 1) & 1, k & 1
                    ixb[pl.ds(wb + k * C, LANES)] = rows[k]
                    wb_ref[pl.ds(wb + k * C, LANES)] = wxy[kx * 2 + ky] * wz[kz]

        def build_and_fire(t, cb, p):
            phase1(t, 0, cb, ixs[p], ws[p], True)

            @pl.loop(1, NLEV)
            def _lev(l):
                phase1(t, l, cb, ixs[p], ws[p], False)
            pltpu.async_copy(scr.at[ixs[p]], rws[p], sems[p])

        def combine(t, p):
            pltpu.make_async_copy(scr.at[ixs[p]], rws[p], sems[p]).wait()

            @pl.loop(0, NLEV)
            def _lev2(l):
                for g in range(G):
                    acc0 = jnp.zeros((LANES,), jnp.float32)
                    acc1 = jnp.zeros((LANES,), jnp.float32)
                    wb = l * (8 * C) + g * LANES
                    for k in range(8):
                        u = rws[p][pl.ds(wb + k * C, LANES)]
                        v0 = plsc.bitcast(u << 16, jnp.float32)
                        v1 = plsc.bitcast(u & jnp.int32(-65536), jnp.float32)
                        wv = ws[p][pl.ds(wb + k * C, LANES)]
                        acc0 = acc0 + v0 * wv
                        acc1 = acc1 + v1 * wv
                    svec = srow + (g * LANES * OCOLS + t * 2 * NLEV) + l * 2
                    plsc.store_scatter(obuf, [svec], acc0)
                    plsc.store_scatter(obuf, [svec + 1], acc1)

        load_coords(0, 0)
        build_and_fire(0, coff(0), 0)

        @pl.loop(0, chunks)
        def _chunk(ch):
            load_coords(ch + 1, (ch + 1) & 1)
            for t in range(4):
                p = t & 1
                np_ = (t + 1) & 1
                if t < 3:
                    build_and_fire(t + 1, coff(ch), np_)
                else:
                    build_and_fire(0, coff(ch + 1), np_)
                combine(t, p)
            base = wid * chunks + ch
            pltpu.sync_copy(obuf, out_hbm.at[pl.ds(base * (C * OCOLS), C * OCOLS)])

        pltpu.make_async_copy(scr.at[ix0], rw0, sem0).wait()

    return sc_encode


def _body_view(t):
    return t[:BODY].reshape(BODY // 128, 128, 2).transpose(0, 2, 1).reshape(-1)


def _tail_tile(t):
    return jnp.pad(t[BODY:], ((0, 128 - (TOTAL - BODY)), (0, 0))).T.reshape(-1)


def kernel(coords, table_xyz, table_xyt, table_yzt, table_xzt):
    n = coords.shape[0]
    ts = (table_xyz, table_xyt, table_yzt, table_xzt)
    coords_v = coords.reshape(n // 128, 128, 4).transpose(0, 2, 1).reshape(-1)
    out, _ = _build(n)(coords_v, *[_body_view(t) for t in ts],
                       *[_tail_tile(t) for t in ts])
    return out.reshape(n, OCOLS)
